# baseline (speedup 1.0000x reference)
import jax
import jax.numpy as jnp
from jax.experimental import pallas as pl
from jax.experimental.pallas import tpu as pltpu

_B, _N, _D, _M = 2, 1024, 16, 16
_BI = 512
_G = 16
_NG = _BI // _G


def _silu(x):
    u = x * 0.5
    return u * jnp.tanh(u) + u


def _egnn_kernel(
    featsbf_ref,
    coorsA_ref,
    coorsXC_ref,
    coorsTi_ref,
    featsP_ref,
    coorsP_ref,
    velP_ref,
    W1t_ref,
    Wstk_ref,
    Wnar_ref,
    Brows_ref,
    bc2_s,
    bv_s,
    hP_ref,
    coorsPn_ref,
    velPn_ref,
):
    feats_bf = featsbf_ref[0]
    coors_all = coorsA_ref[0]
    XC = coorsXC_ref[0]
    xi = coorsTi_ref[0]
    P = featsP_ref[0]
    coorsP = coorsP_ref[0]
    velP = velP_ref[0]

    f32 = jnp.float32
    bf16 = jnp.bfloat16
    hi = jax.lax.Precision.HIGHEST

    def mm(a, b, prec=None):
        return jax.lax.dot_general(
            a, b, (((1,), (0,)), ((), ())),
            preferred_element_type=f32, precision=prec)


    dist = None
    for a in range(3):
        d = coors_all[:, a : a + 1] - xi[a : a + 1, :]
        dist = d * d if dist is None else dist + d * d
    dist_h = dist.astype(bf16)
    dist_l = (dist - dist_h.astype(f32)).astype(bf16)

    ai_half = (mm(P, Wstk_ref[2], hi) * 0.5).astype(bf16)

    ones_col = jnp.ones((_N, 1), dtype=bf16)
    W1t = W1t_ref[...]
    W2bd = Wstk_ref[0].astype(bf16)
    W3bd = Wstk_ref[1].astype(bf16)
    b2half = (Brows_ref[0:1, :] * 0.5).astype(bf16)
    b3half = (Brows_ref[1:2, :] * 0.5).astype(bf16)

    msum_rows = []
    s0_rows = []
    sx_rows = [[], [], []]
    def silu_biased(z, bhalf):
        u = z * jnp.asarray(0.5, z.dtype) + bhalf
        return u * jnp.tanh(u) + u

    for g in range(_NG):
        sl = slice(_G * g, _G * (g + 1))
        X = jnp.concatenate(
            [feats_bf, dist_h[:, sl], dist_l[:, sl], ones_col], axis=1
        )
        m1 = silu_biased(mm(X, W1t).astype(bf16), ai_half[g : g + 1, :])
        m2 = silu_biased(mm(m1, W2bd).astype(bf16), b2half)
        msum_rows.append(
            jnp.sum(m2, axis=0, keepdims=True, dtype=f32)
        )
        p3 = silu_biased(mm(m2, W3bd).astype(bf16), b3half)
        T = mm(XC, p3)
        s0_rows.append(T[0:1, :])
        for a in range(3):
            sx_rows[a].append(T[a + 1 : a + 2, :])

    Msum = jnp.concatenate(msum_rows, axis=0)
    S0 = jnp.concatenate(s0_rows, axis=0)
    W4bd = Wnar_ref[0]
    bc2 = bc2_s[0, 0]

    SC = mm(S0, W4bd, hi) + (_N * bc2)
    sumx = jnp.sum(coors_all, axis=0, keepdims=True)

    gate = mm(P, Wnar_ref[1], hi) + bv_s[0, 0]

    inv_n = 1.0 / _N
    for a in range(3):
        SXa = jnp.concatenate(sx_rows[a], axis=0)
        CXa = mm(SXa, W4bd, hi) + bc2 * sumx[0, a]
        agg_a = (SC * coorsP[a] - CXa) * inv_n
        vel_a = gate * velP[a] + agg_a
        velPn_ref[0, a] = vel_a
        coorsPn_ref[0, a] = coorsP[a] + vel_a

    n1 = _silu(
        mm(P, Wstk_ref[3], hi)
        + mm(Msum, Wstk_ref[4], hi)
        + Brows_ref[2:3, :]
    )
    h2 = mm(n1, Wstk_ref[5], hi) + Brows_ref[3:4, :]
    hP_ref[0] = P + h2


@jax.jit
def kernel(feats, coors, vel, We1, be1, We2, be2, Wc1, bc1, Wc2, bc2,
           Wv, bv, Wn1, bn1, Wn2, bn2):
    f32 = jnp.float32
    bf16 = jnp.bfloat16
    eye = jnp.eye(_G, dtype=f32)

    feats_bf = feats.astype(bf16)
    coorsT = jnp.transpose(coors, (0, 2, 1))
    coorsXC = jnp.concatenate(
        [jnp.ones((_B, 1, _N), f32), coorsT], axis=1
    ).astype(bf16)
    featsP = jnp.reshape(feats, (_B, _N // _G, _G * _D))
    coorsP = jnp.reshape(coorsT, (_B, 3, _N // _G, _G))
    velP = jnp.reshape(jnp.transpose(vel, (0, 2, 1)), (_B, 3, _N // _G, _G))

    wd = We1[2 * _D, :]
    W1t = jnp.concatenate(
        [
            jnp.tile(We1[_D : 2 * _D, :], (1, _G)),
            jnp.kron(eye, wd[None, :]),
            jnp.kron(eye, wd[None, :]),
            jnp.tile(be1[None, :], (1, _G)),
        ],
        axis=0,
    ).astype(bf16)
    Wsq = jnp.stack([We2, Wc1, We1[:_D, :], Wn1[:_D, :], Wn1[_D:, :], Wn2])
    Wstk = jnp.reshape(
        jnp.einsum("gh,kij->kgihj", eye, Wsq),
        (6, _G * _M, _G * _M),
    )
    Wnr = jnp.stack([Wc2, jnp.reshape(Wv, (_D, 1))])
    Wnar = jnp.reshape(
        jnp.einsum("gh,kij->kgihj", eye, Wnr),
        (2, _G * _M, _G),
    )
    Brows = jnp.tile(jnp.stack([be2, bc1, bn1, bn2]), (1, _G))

    grid = (_B, _N // _BI)
    ng = _BI // _G

    full = lambda shape: pl.BlockSpec(shape, lambda b, i: tuple(0 for _ in shape))
    smem = lambda shape: pl.BlockSpec(
        shape, lambda b, i: tuple(0 for _ in shape), memory_space=pltpu.SMEM
    )

    out_shapes = (
        jax.ShapeDtypeStruct((_B, _N // _G, _G * _D), f32),
        jax.ShapeDtypeStruct((_B, 3, _N // _G, _G), f32),
        jax.ShapeDtypeStruct((_B, 3, _N // _G, _G), f32),
    )

    hP, coorsPn, velPn = pl.pallas_call(
        _egnn_kernel,
        grid=grid,
        in_specs=[
            pl.BlockSpec((1, _N, _D), lambda b, i: (b, 0, 0)),
            pl.BlockSpec((1, _N, 3), lambda b, i: (b, 0, 0)),
            pl.BlockSpec((1, 4, _N), lambda b, i: (b, 0, 0)),
            pl.BlockSpec((1, 3, _BI), lambda b, i: (b, 0, i)),
            pl.BlockSpec((1, ng, _G * _D), lambda b, i: (b, i, 0)),
            pl.BlockSpec((1, 3, ng, _G), lambda b, i: (b, 0, i, 0)),
            pl.BlockSpec((1, 3, ng, _G), lambda b, i: (b, 0, i, 0)),
            full((2 * _D + 17, _G * _M)),
            full((6, _G * _M, _G * _M)),
            full((2, _G * _M, _G)),
            full((4, _G * _M)),
            smem((1, 1)),
            smem((1, 1)),
        ],
        out_specs=[
            pl.BlockSpec((1, ng, _G * _D), lambda b, i: (b, i, 0)),
            pl.BlockSpec((1, 3, ng, _G), lambda b, i: (b, 0, i, 0)),
            pl.BlockSpec((1, 3, ng, _G), lambda b, i: (b, 0, i, 0)),
        ],
        out_shape=out_shapes,
    )(
        feats_bf, coors, coorsXC, coorsT, featsP, coorsP, velP,
        W1t, Wstk, Wnar, Brows,
        jnp.reshape(bc2, (1, 1)), jnp.reshape(bv, (1, 1)),
    )

    h_new = jnp.reshape(hP, (_B, _N, _D))
    coors_new = jnp.transpose(jnp.reshape(coorsPn, (_B, 3, _N)), (0, 2, 1))
    vel_new = jnp.transpose(jnp.reshape(velPn, (_B, 3, _N)), (0, 2, 1))
    return (h_new, coors_new, vel_new)


# speedup vs baseline: 1.0117x; 1.0117x over previous
import jax
import jax.numpy as jnp
from jax.experimental import pallas as pl
from jax.experimental.pallas import tpu as pltpu

_B, _N, _D, _M = 2, 1024, 16, 16
_BI = 512
_G = 16
_NG = _BI // _G


def _silu(x):
    u = x * 0.5
    return u * jnp.tanh(u) + u


def _egnn_kernel(
    featsA_ref,
    coorsA_ref,
    coorsTf_ref,
    coorsTi_ref,
    featsP_ref,
    coorsP_ref,
    velP_ref,
    W1t_ref,
    Wstk_ref,
    Wnar_ref,
    Brows_ref,
    bc2_s,
    bv_s,
    hP_ref,
    coorsPn_ref,
    velPn_ref,
):
    f32_ = jnp.float32
    feats_bf = featsA_ref[0].astype(jnp.bfloat16)
    coors_all = coorsA_ref[0]
    XC = jnp.concatenate(
        [jnp.ones((1, _N), jnp.bfloat16), coorsTf_ref[0].astype(jnp.bfloat16)],
        axis=0,
    )
    xi = coorsTi_ref[0]
    P = featsP_ref[0]
    coorsP = coorsP_ref[0]
    velP = velP_ref[0]

    f32 = jnp.float32
    bf16 = jnp.bfloat16
    hi = jax.lax.Precision.HIGHEST

    def mm(a, b, prec=None):
        return jax.lax.dot_general(
            a, b, (((1,), (0,)), ((), ())),
            preferred_element_type=f32, precision=prec)


    dist = None
    for a in range(3):
        d = coors_all[:, a : a + 1] - xi[a : a + 1, :]
        dist = d * d if dist is None else dist + d * d
    dist_h = dist.astype(bf16)
    dist_l = (dist - dist_h.astype(f32)).astype(bf16)

    ai_half = (mm(P, Wstk_ref[2], hi) * 0.5).astype(bf16)

    ones_col = jnp.ones((_N, 1), dtype=bf16)
    W1t = W1t_ref[...]
    W2bd = Wstk_ref[0].astype(bf16)
    W3bd = Wstk_ref[1].astype(bf16)
    b2half = (Brows_ref[0:1, :] * 0.5).astype(bf16)
    b3half = (Brows_ref[1:2, :] * 0.5).astype(bf16)

    msum_rows = []
    s0_rows = []
    sx_rows = [[], [], []]
    def silu_biased(z, bhalf):
        u = z * jnp.asarray(0.5, z.dtype) + bhalf
        return u * jnp.tanh(u) + u

    for g in range(_NG):
        sl = slice(_G * g, _G * (g + 1))
        X = jnp.concatenate(
            [feats_bf, dist_h[:, sl], dist_l[:, sl], ones_col], axis=1
        )
        m1 = silu_biased(mm(X, W1t).astype(bf16), ai_half[g : g + 1, :])
        m2 = silu_biased(mm(m1, W2bd).astype(bf16), b2half)
        msum_rows.append(
            jnp.sum(m2, axis=0, keepdims=True, dtype=f32)
        )
        p3 = silu_biased(mm(m2, W3bd).astype(bf16), b3half)
        T = mm(XC, p3)
        s0_rows.append(T[0:1, :])
        for a in range(3):
            sx_rows[a].append(T[a + 1 : a + 2, :])

    Msum = jnp.concatenate(msum_rows, axis=0)
    S0 = jnp.concatenate(s0_rows, axis=0)
    W4bd = Wnar_ref[0]
    bc2 = bc2_s[0, 0]

    SC = mm(S0, W4bd, hi) + (_N * bc2)
    sumx = jnp.sum(coors_all, axis=0, keepdims=True)

    gate = mm(P, Wnar_ref[1], hi) + bv_s[0, 0]

    inv_n = 1.0 / _N
    for a in range(3):
        SXa = jnp.concatenate(sx_rows[a], axis=0)
        CXa = mm(SXa, W4bd, hi) + bc2 * sumx[0, a]
        agg_a = (SC * coorsP[a] - CXa) * inv_n
        vel_a = gate * velP[a] + agg_a
        velPn_ref[0, a] = vel_a
        coorsPn_ref[0, a] = coorsP[a] + vel_a

    n1 = _silu(
        mm(P, Wstk_ref[3], hi)
        + mm(Msum, Wstk_ref[4], hi)
        + Brows_ref[2:3, :]
    )
    h2 = mm(n1, Wstk_ref[5], hi) + Brows_ref[3:4, :]
    hP_ref[0] = P + h2


@jax.jit
def kernel(feats, coors, vel, We1, be1, We2, be2, Wc1, bc1, Wc2, bc2,
           Wv, bv, Wn1, bn1, Wn2, bn2):
    f32 = jnp.float32
    bf16 = jnp.bfloat16
    eye = jnp.eye(_G, dtype=f32)

    coorsT = jnp.transpose(coors, (0, 2, 1))
    featsP = jnp.reshape(feats, (_B, _N // _G, _G * _D))
    coorsP = jnp.reshape(coorsT, (_B, 3, _N // _G, _G))
    velP = jnp.reshape(jnp.transpose(vel, (0, 2, 1)), (_B, 3, _N // _G, _G))

    wd = We1[2 * _D, :]
    W1t = jnp.concatenate(
        [
            jnp.tile(We1[_D : 2 * _D, :], (1, _G)),
            jnp.kron(eye, wd[None, :]),
            jnp.kron(eye, wd[None, :]),
            jnp.tile(be1[None, :], (1, _G)),
        ],
        axis=0,
    ).astype(bf16)
    Wsq = jnp.stack([We2, Wc1, We1[:_D, :], Wn1[:_D, :], Wn1[_D:, :], Wn2])
    Wstk = jnp.reshape(
        jnp.einsum("gh,kij->kgihj", eye, Wsq),
        (6, _G * _M, _G * _M),
    )
    Wnr = jnp.stack([Wc2, jnp.reshape(Wv, (_D, 1))])
    Wnar = jnp.reshape(
        jnp.einsum("gh,kij->kgihj", eye, Wnr),
        (2, _G * _M, _G),
    )
    Brows = jnp.tile(jnp.stack([be2, bc1, bn1, bn2]), (1, _G))

    grid = (_B, _N // _BI)
    ng = _BI // _G

    full = lambda shape: pl.BlockSpec(shape, lambda b, i: tuple(0 for _ in shape))
    smem = lambda shape: pl.BlockSpec(
        shape, lambda b, i: tuple(0 for _ in shape), memory_space=pltpu.SMEM
    )

    out_shapes = (
        jax.ShapeDtypeStruct((_B, _N // _G, _G * _D), f32),
        jax.ShapeDtypeStruct((_B, 3, _N // _G, _G), f32),
        jax.ShapeDtypeStruct((_B, 3, _N // _G, _G), f32),
    )

    hP, coorsPn, velPn = pl.pallas_call(
        _egnn_kernel,
        grid=grid,
        in_specs=[
            pl.BlockSpec((1, _N, _D), lambda b, i: (b, 0, 0)),
            pl.BlockSpec((1, _N, 3), lambda b, i: (b, 0, 0)),
            pl.BlockSpec((1, 3, _N), lambda b, i: (b, 0, 0)),
            pl.BlockSpec((1, 3, _BI), lambda b, i: (b, 0, i)),
            pl.BlockSpec((1, ng, _G * _D), lambda b, i: (b, i, 0)),
            pl.BlockSpec((1, 3, ng, _G), lambda b, i: (b, 0, i, 0)),
            pl.BlockSpec((1, 3, ng, _G), lambda b, i: (b, 0, i, 0)),
            full((2 * _D + 17, _G * _M)),
            full((6, _G * _M, _G * _M)),
            full((2, _G * _M, _G)),
            full((4, _G * _M)),
            smem((1, 1)),
            smem((1, 1)),
        ],
        out_specs=[
            pl.BlockSpec((1, ng, _G * _D), lambda b, i: (b, i, 0)),
            pl.BlockSpec((1, 3, ng, _G), lambda b, i: (b, 0, i, 0)),
            pl.BlockSpec((1, 3, ng, _G), lambda b, i: (b, 0, i, 0)),
        ],
        out_shape=out_shapes,
    )(
        feats, coors, coorsT, coorsT, featsP, coorsP, velP,
        W1t, Wstk, Wnar, Brows,
        jnp.reshape(bc2, (1, 1)), jnp.reshape(bv, (1, 1)),
    )

    h_new = jnp.reshape(hP, (_B, _N, _D))
    coors_new = jnp.transpose(jnp.reshape(coorsPn, (_B, 3, _N)), (0, 2, 1))
    vel_new = jnp.transpose(jnp.reshape(velPn, (_B, 3, _N)), (0, 2, 1))
    return (h_new, coors_new, vel_new)


# speedup vs baseline: 1.0131x; 1.0013x over previous
import jax
import jax.numpy as jnp
from jax.experimental import pallas as pl
from jax.experimental.pallas import tpu as pltpu

_B, _N, _D, _M = 2, 1024, 16, 16
_BI = 512
_G = 16
_NG = _BI // _G


def _silu(x):
    u = x * 0.5
    return u * jnp.tanh(u) + u


def _egnn_kernel(
    featsA_ref,
    coorsA_ref,
    coorsTf_ref,
    coorsTi_ref,
    featsP_ref,
    coorsP_ref,
    velP_ref,
    W1t_ref,
    Wstk_ref,
    Wnar_ref,
    Brows_ref,
    bc2_s,
    bv_s,
    hP_ref,
    coorsPn_ref,
    velPn_ref,
):
    feats_bf = featsA_ref[0].astype(jnp.bfloat16)
    coors_all = coorsA_ref[0]
    XC = jnp.concatenate(
        [jnp.ones((1, _N), jnp.bfloat16), coorsTf_ref[0].astype(jnp.bfloat16)],
        axis=0,
    )
    xi = coorsTi_ref[0]
    P = featsP_ref[0]
    coorsP = coorsP_ref[0]
    velP = velP_ref[0]

    f32 = jnp.float32
    bf16 = jnp.bfloat16
    hi = jax.lax.Precision.HIGHEST

    def mm(a, b, prec=None):
        return jax.lax.dot_general(
            a, b, (((1,), (0,)), ((), ())),
            preferred_element_type=f32, precision=prec)


    dist = None
    for a in range(3):
        d = coors_all[:, a : a + 1] - xi[a : a + 1, :]
        dist = d * d if dist is None else dist + d * d
    dist_h = dist.astype(bf16)
    dist_l = (dist - dist_h.astype(f32)).astype(bf16)

    ai_half = (mm(P, Wstk_ref[2], hi) * 0.5).astype(bf16)

    ones_col = jnp.ones((_N, 1), dtype=bf16)
    W1t = W1t_ref[...]
    W2bd = Wstk_ref[0].astype(bf16)
    W3bd = Wstk_ref[1].astype(bf16)
    b2half = (Brows_ref[0:1, :] * 0.5).astype(bf16)
    b3half = (Brows_ref[1:2, :] * 0.5).astype(bf16)

    msum_rows = []
    s0_rows = []
    sx_rows = [[], [], []]
    def silu_biased(z, bhalf):
        u = z * jnp.asarray(0.5, z.dtype) + bhalf
        return u * jnp.tanh(u) + u

    for g in range(_NG):
        sl = slice(_G * g, _G * (g + 1))
        X = jnp.concatenate(
            [feats_bf, dist_h[:, sl], dist_l[:, sl], ones_col], axis=1
        )
        m1 = silu_biased(mm(X, W1t).astype(bf16), ai_half[g : g + 1, :])
        m2 = silu_biased(mm(m1, W2bd).astype(bf16), b2half)
        msum_rows.append(
            jnp.sum(m2, axis=0, keepdims=True, dtype=f32)
        )
        p3 = silu_biased(mm(m2, W3bd).astype(bf16), b3half)
        T = mm(XC, p3)
        s0_rows.append(T[0:1, :])
        for a in range(3):
            sx_rows[a].append(T[a + 1 : a + 2, :])

    Msum = jnp.concatenate(msum_rows, axis=0)
    S0 = jnp.concatenate(s0_rows, axis=0)
    W4bd = Wnar_ref[0]
    bc2 = bc2_s[0, 0]

    SC = mm(S0, W4bd, hi) + (_N * bc2)
    sumx = jnp.sum(coors_all, axis=0, keepdims=True)

    gate = mm(P, Wnar_ref[1], hi) + bv_s[0, 0]

    inv_n = 1.0 / _N
    for a in range(3):
        SXa = jnp.concatenate(sx_rows[a], axis=0)
        CXa = mm(SXa, W4bd, hi) + bc2 * sumx[0, a]
        agg_a = (SC * coorsP[a] - CXa) * inv_n
        vel_a = gate * velP[a] + agg_a
        velPn_ref[0, a] = vel_a
        coorsPn_ref[0, a] = coorsP[a] + vel_a

    n1 = _silu(
        mm(P, Wstk_ref[3], hi)
        + mm(Msum, Wstk_ref[4], hi)
        + Brows_ref[2:3, :]
    )
    h2 = mm(n1, Wstk_ref[5], hi) + Brows_ref[3:4, :]
    hP_ref[0] = P + h2


@jax.jit
def kernel(feats, coors, vel, We1, be1, We2, be2, Wc1, bc1, Wc2, bc2,
           Wv, bv, Wn1, bn1, Wn2, bn2):
    f32 = jnp.float32
    bf16 = jnp.bfloat16
    eye = jnp.eye(_G, dtype=f32)

    coorsT = jnp.transpose(coors, (0, 2, 1))
    featsP = jnp.reshape(feats, (_B, _N // _G, _G * _D))
    coorsP = jnp.reshape(coorsT, (_B, 3, _N // _G, _G))
    velP = jnp.reshape(jnp.transpose(vel, (0, 2, 1)), (_B, 3, _N // _G, _G))

    wd = We1[2 * _D, :]
    W1t = jnp.concatenate(
        [
            jnp.tile(We1[_D : 2 * _D, :], (1, _G)),
            jnp.kron(eye, wd[None, :]),
            jnp.kron(eye, wd[None, :]),
            jnp.tile(be1[None, :], (1, _G)),
        ],
        axis=0,
    ).astype(bf16)
    Wsq = jnp.stack([We2, Wc1, We1[:_D, :], Wn1[:_D, :], Wn1[_D:, :], Wn2])
    Wstk = jnp.reshape(
        jnp.einsum("gh,kij->kgihj", eye, Wsq),
        (6, _G * _M, _G * _M),
    )
    Wnr = jnp.stack([Wc2, jnp.reshape(Wv, (_D, 1))])
    Wnar = jnp.reshape(
        jnp.einsum("gh,kij->kgihj", eye, Wnr),
        (2, _G * _M, _G),
    )
    Brows = jnp.tile(jnp.stack([be2, bc1, bn1, bn2]), (1, _G))

    grid = (_B, _N // _BI)
    ng = _BI // _G

    full = lambda shape: pl.BlockSpec(shape, lambda b, i: tuple(0 for _ in shape))
    smem = lambda shape: pl.BlockSpec(
        shape, lambda b, i: tuple(0 for _ in shape), memory_space=pltpu.SMEM
    )

    out_shapes = (
        jax.ShapeDtypeStruct((_B, _N // _G, _G * _D), f32),
        jax.ShapeDtypeStruct((_B, 3, _N // _G, _G), f32),
        jax.ShapeDtypeStruct((_B, 3, _N // _G, _G), f32),
    )

    hP, coorsPn, velPn = pl.pallas_call(
        _egnn_kernel,
        grid=grid,
        in_specs=[
            pl.BlockSpec((1, _N, _D), lambda b, i: (b, 0, 0)),
            pl.BlockSpec((1, _N, 3), lambda b, i: (b, 0, 0)),
            pl.BlockSpec((1, 3, _N), lambda b, i: (b, 0, 0)),
            pl.BlockSpec((1, 3, _BI), lambda b, i: (b, 0, i)),
            pl.BlockSpec((1, ng, _G * _D), lambda b, i: (b, i, 0)),
            pl.BlockSpec((1, 3, ng, _G), lambda b, i: (b, 0, i, 0)),
            pl.BlockSpec((1, 3, ng, _G), lambda b, i: (b, 0, i, 0)),
            full((2 * _D + 17, _G * _M)),
            full((6, _G * _M, _G * _M)),
            full((2, _G * _M, _G)),
            full((4, _G * _M)),
            smem((1, 1)),
            smem((1, 1)),
        ],
        out_specs=[
            pl.BlockSpec((1, ng, _G * _D), lambda b, i: (b, i, 0)),
            pl.BlockSpec((1, 3, ng, _G), lambda b, i: (b, 0, i, 0)),
            pl.BlockSpec((1, 3, ng, _G), lambda b, i: (b, 0, i, 0)),
        ],
        out_shape=out_shapes,
    )(
        feats, coors, coorsT, coorsT, featsP, coorsP, velP,
        W1t, Wstk, Wnar, Brows,
        jnp.reshape(bc2, (1, 1)), jnp.reshape(bv, (1, 1)),
    )

    h_new = jnp.reshape(hP, (_B, _N, _D))
    coors_new = jnp.transpose(jnp.reshape(coorsPn, (_B, 3, _N)), (0, 2, 1))
    vel_new = jnp.transpose(jnp.reshape(velPn, (_B, 3, _N)), (0, 2, 1))
    return (h_new, coors_new, vel_new)
